# Initial kernel scaffold; baseline (speedup 1.0000x reference)
import jax
import jax.numpy as jnp
from jax.experimental import pallas as pl


def kernel(edge_index, features, lin_W, lin_b, gcn1_W, gcn1_b, gcn2_W, gcn2_b, lin1_W, lin1_b):
    raise NotImplementedError("write your pallas kernel here")


# speedup vs baseline: 7.4216x; 7.4216x over previous
import functools

import jax
import jax.numpy as jnp
from jax import lax
from jax.experimental import pallas as pl
from jax.experimental.pallas import tpu as pltpu
from jax.experimental.pallas import tpu_sc as plsc

N = 10000
D = 128
T = 2
M = 64
E = 160000
L = 16
NC = 2
NS = 16
CH = 128
NROWS = 1280
EP = NROWS * CH
TRASH = 3 * N
ACC_ROWS = 30080
ACC_PT = ACC_ROWS // NS
ND = 10240
DEG_PT = ND // NS
RB = 2000
GRID = N // RB

_sc_params = pltpu.CompilerParams(use_tc_tiling_on_sc=False,
                                  needs_layout_passes=False)


def _sim_body(dstack, fhat, srcp, dstp,
              idx0, idx1, idx2, degacc,
              sbuf, dbuf, drows, frows, i0b, i1b, i2b, valb, zdeg, dacc,
              sem_a, sem_b):
    c = lax.axis_index("c")
    s = lax.axis_index("s")
    wid = s * NC + c
    iota = lax.iota(jnp.int32, L)
    zf = jnp.zeros((L,), jnp.float32)

    def zr(r, _):
        zdeg[r, :] = zf
        return 0
    lax.fori_loop(0, DEG_PT, zr, 0)
    pltpu.sync_copy(zdeg, dacc.at[pl.ds(s * DEG_PT, DEG_PT)])
    plsc.subcore_barrier()

    rows_pt = NROWS // (NC * NS)

    def chunk(j, _):
        row = wid * rows_pt + j
        pltpu.sync_copy(srcp.at[row, 0], sbuf)
        pltpu.sync_copy(dstp.at[row, 0], dbuf)
        cp1 = pltpu.async_copy(dstack.at[sbuf], drows, sem_a)
        cp2 = pltpu.async_copy(fhat.at[dbuf], frows, sem_b)
        cp1.wait()
        cp2.wait()
        for g in range(CH // L):
            rowv = g * L + iota
            dstv = dbuf[pl.ds(g * L, L)]

            def dot_step(jj, acc):
                colv = jnp.full((L,), jj, jnp.int32)
                dv1 = plsc.load_gather(drows, [rowv, colv])
                dv2 = plsc.load_gather(drows, [rowv, colv + D])
                fv = plsc.load_gather(frows, [rowv, colv])
                return (acc[0] + dv1 * fv, acc[1] + dv2 * fv)

            acc1, acc2 = lax.fori_loop(0, D, dot_step, (zf, zf))
            eid = row * CH + g * L + iota
            valid = eid < E
            w1 = valid & (acc1 > 0.0)
            w2 = valid & (acc2 > 0.0)
            i0b[pl.ds(g * L, L)] = jnp.where(valid, dstv, TRASH)
            i1b[pl.ds(g * L, L)] = jnp.where(w1, dstv + N, TRASH)
            i2b[pl.ds(g * L, L)] = jnp.where(w2, dstv + 2 * N, TRASH)
            one = jnp.ones((L,), jnp.float32)
            plsc.store_scatter(valb, [rowv, jnp.full((L,), 0, jnp.int32)],
                               jnp.where(valid, one, 0.0))
            plsc.store_scatter(valb, [rowv, jnp.full((L,), 1, jnp.int32)],
                               jnp.where(w1, one, 0.0))
            plsc.store_scatter(valb, [rowv, jnp.full((L,), 2, jnp.int32)],
                               jnp.where(w2, one, 0.0))
        pltpu.sync_copy(valb, dacc.at[dbuf], add=True)
        pltpu.sync_copy(i0b, idx0.at[row, 0])
        pltpu.sync_copy(i1b, idx1.at[row, 0])
        pltpu.sync_copy(i2b, idx2.at[row, 0])
        return 0

    lax.fori_loop(0, rows_pt, chunk, 0)
    plsc.subcore_barrier()
    pltpu.sync_copy(dacc.at[pl.ds(s * DEG_PT, DEG_PT)],
                    degacc.at[pl.ds(c * ND + s * DEG_PT, DEG_PT)])


def _make_sim_kernel():
  return functools.partial(
    pl.kernel,
    out_type=[
        jax.ShapeDtypeStruct((NROWS, 1, CH), jnp.int32),
        jax.ShapeDtypeStruct((NROWS, 1, CH), jnp.int32),
        jax.ShapeDtypeStruct((NROWS, 1, CH), jnp.int32),
        jax.ShapeDtypeStruct((NC * ND, L), jnp.float32),
    ],
    mesh=plsc.VectorSubcoreMesh(core_axis_name="c", subcore_axis_name="s",
                                num_cores=NC, num_subcores=NS),
    scratch_types=[
        pltpu.VMEM((CH,), jnp.int32),
        pltpu.VMEM((CH,), jnp.int32),
        pltpu.VMEM((CH, 2 * D), jnp.float32),
        pltpu.VMEM((CH, D), jnp.float32),
        pltpu.VMEM((CH,), jnp.int32),
        pltpu.VMEM((CH,), jnp.int32),
        pltpu.VMEM((CH,), jnp.int32),
        pltpu.VMEM((CH, L), jnp.float32),
        pltpu.VMEM((DEG_PT, L), jnp.float32),
        pltpu.VMEM_SHARED((ND, L), jnp.float32),
        pltpu.SemaphoreType.DMA,
        pltpu.SemaphoreType.DMA,
    ],
    compiler_params=_sc_params,
  )(_sim_body)


def _agg_body(table, srcp, idx0, idx1, idx2,
              macc,
              sbuf, i0b, i1b, i2b, rows, acc, sem):
    c = lax.axis_index("c")
    s = lax.axis_index("s")
    zf = jnp.zeros((L,), jnp.float32)

    def zr2(r, _):
        for q in range(M // L):
            rows[r, pl.ds(q * L, L)] = zf
        return 0
    lax.fori_loop(0, CH, zr2, 0)
    nfull = ACC_PT // CH
    rem = ACC_PT - nfull * CH
    for q in range(nfull):
        pltpu.sync_copy(rows, acc.at[pl.ds(s * ACC_PT + q * CH, CH)])
    pltpu.sync_copy(rows.at[pl.ds(0, rem)],
                    acc.at[pl.ds(s * ACC_PT + nfull * CH, rem)])
    plsc.subcore_barrier()

    rows_pt = NROWS // NS

    coff = c * N

    def chunk(j, _):
        row = s * rows_pt + j
        pltpu.sync_copy(srcp.at[row, 0], sbuf)
        pltpu.sync_copy(idx0.at[row, 0], i0b)
        pltpu.sync_copy(idx1.at[row, 0], i1b)
        pltpu.sync_copy(idx2.at[row, 0], i2b)
        for g in range(CH // L):
            sbuf[pl.ds(g * L, L)] = sbuf[pl.ds(g * L, L)] + coff
        pltpu.async_copy(table.at[sbuf], rows, sem).wait()
        pltpu.sync_copy(rows, acc.at[i0b], add=True)
        pltpu.sync_copy(rows, acc.at[i1b], add=True)
        pltpu.sync_copy(rows, acc.at[i2b], add=True)
        return 0

    lax.fori_loop(0, rows_pt, chunk, 0)
    plsc.subcore_barrier()
    pltpu.sync_copy(acc.at[pl.ds(s * ACC_PT, ACC_PT)],
                    macc.at[pl.ds(c * ACC_ROWS + s * ACC_PT, ACC_PT)])


def _make_agg_kernel():
  return functools.partial(
    pl.kernel,
    out_type=[jax.ShapeDtypeStruct((NC * ACC_ROWS, M), jnp.float32)],
    mesh=plsc.VectorSubcoreMesh(core_axis_name="c", subcore_axis_name="s",
                                num_cores=NC, num_subcores=NS),
    scratch_types=[
        pltpu.VMEM((CH,), jnp.int32),
        pltpu.VMEM((CH,), jnp.int32),
        pltpu.VMEM((CH,), jnp.int32),
        pltpu.VMEM((CH,), jnp.int32),
        pltpu.VMEM((CH, M), jnp.float32),
        pltpu.VMEM_SHARED((ACC_ROWS, M), jnp.float32),
        pltpu.SemaphoreType.DMA,
    ],
    compiler_params=_sc_params,
  )(_agg_body)


def _full(shape):
    return pl.BlockSpec(shape, lambda i: tuple(0 for _ in shape))


def _rows(shape):
    return pl.BlockSpec(shape, lambda i: (i,) + tuple(0 for _ in shape[1:]))


def _mid(shape, off):
    return pl.BlockSpec(shape, lambda i, _o=off: (0, _o + i, 0))


def _rows3(shape):
    return pl.BlockSpec(shape, lambda i: (0, i, 0))


def _prep_body(f_ref, linw_ref, linb_ref, w1_ref,
               fhat_ref, dstack_ref, xw1_ref):
    f = f_ref[...]
    nf = jnp.maximum(jnp.sqrt(jnp.sum(f * f, axis=1, keepdims=True)), 1e-8)
    fhat = f / nf
    fhat_ref[...] = fhat
    for i in range(T):
        ti = jnp.dot(f, linw_ref[i], preferred_element_type=jnp.float32) \
            + linb_ref[i:i + 1, :]
        nt = jnp.maximum(jnp.sqrt(jnp.sum(ti * ti, axis=1, keepdims=True)), 1e-8)
        dstack_ref[:, i * D:(i + 1) * D] = ti / nt - fhat
    xw1_ref[...] = jnp.dot(jnp.maximum(f, 0.0), w1_ref[...],
                           preferred_element_type=jnp.float32)


def _tc_prep(f, lin_W, lin_b, gcn1_W):
    return pl.pallas_call(
        _prep_body,
        grid=(GRID,),
        in_specs=[_rows((RB, D)), _full((T, D, D)), _full((T, D)),
                  _full((D, D))],
        out_specs=[_rows((RB, D)), _rows((RB, 2 * D)), _rows((RB, D))],
        out_shape=[jax.ShapeDtypeStruct((N, D), jnp.float32),
                   jax.ShapeDtypeStruct((N, 2 * D), jnp.float32),
                   jax.ShapeDtypeStruct((N, D), jnp.float32)],
    )(f, lin_W, lin_b, gcn1_W)


def _dinvs(degacc):
    dsum = degacc[0] + degacc[1]
    indeg = dsum[:, 0:1]
    dinv0 = lax.rsqrt(indeg + 1.0)
    dinv1 = lax.rsqrt(dsum[:, 1:2] + 2.0)
    dinv2 = lax.rsqrt(dsum[:, 2:3] + 2.0)
    return dinv0, dinv1, dinv2


def _p1_body(degacc_ref, xw1_ref, p1_ref):
    dinv0, _, _ = _dinvs(degacc_ref[...])
    p1 = dinv0 * xw1_ref[...]
    p1_ref[0] = p1[:, :M]
    p1_ref[1] = p1[:, M:]


def _tc_p1(degacc, xw1):
    return pl.pallas_call(
        _p1_body,
        grid=(GRID,),
        in_specs=[_rows3((NC, RB, L)), _rows((RB, D))],
        out_specs=[_rows3((NC, RB, M))],
        out_shape=[jax.ShapeDtypeStruct((NC, N, M), jnp.float32)],
    )(degacc, xw1)


def _mcat(m_ref):
    return jnp.concatenate([m_ref[0], m_ref[1]], axis=1)


def _mid_body(m0_ref, m1_ref, m2_ref, degacc_ref, xw1_ref,
              b1_ref, w2_ref, p2_ref, xw2_ref):
    dinv0, dinv1, dinv2 = _dinvs(degacc_ref[...])
    xw1 = xw1_ref[...]
    b1 = b1_ref[...]
    w2 = w2_ref[...]
    out1_0 = dinv0 * _mcat(m0_ref) + (dinv0 * dinv0) * xw1 + b1
    out1_1 = dinv1 * _mcat(m1_ref) + (dinv1 * (dinv0 + dinv1)) * xw1 + b1
    out1_2 = dinv2 * _mcat(m2_ref) + (dinv2 * (dinv0 + dinv2)) * xw1 + b1
    xw2_0 = jnp.dot(jnp.maximum(out1_0, 0.0), w2,
                    preferred_element_type=jnp.float32)
    xw2_ref[0] = xw2_0
    xw2_ref[1] = jnp.dot(jnp.maximum(out1_1, 0.0), w2,
                         preferred_element_type=jnp.float32)
    xw2_ref[2] = jnp.dot(jnp.maximum(out1_2, 0.0), w2,
                         preferred_element_type=jnp.float32)
    p2 = dinv0 * xw2_0
    p2_ref[0] = p2[:, :M]
    p2_ref[1] = p2[:, M:]


def _tc_mid(macc1, degacc, xw1, b1, w2):
    return pl.pallas_call(
        _mid_body,
        grid=(GRID,),
        in_specs=[_mid((NC, RB, M), 0), _mid((NC, RB, M), GRID),
                  _mid((NC, RB, M), 2 * GRID),
                  _rows3((NC, RB, L)), _rows((RB, D)),
                  _full((1, D)), _full((D, D))],
        out_specs=[_rows3((NC, RB, M)), _rows3((3, RB, D))],
        out_shape=[jax.ShapeDtypeStruct((NC, N, M), jnp.float32),
                   jax.ShapeDtypeStruct((3, N, D), jnp.float32)],
    )(macc1, macc1, macc1, degacc, xw1, b1, w2)


def _fin1_body(q0_ref, q1_ref, q2_ref, degacc_ref, xw2_ref,
               b2_ref, l1w_ref, l1b_ref, out20_ref, h_ref):
    dinv0, dinv1, dinv2 = _dinvs(degacc_ref[...])
    b2 = b2_ref[...]
    xw2_0 = xw2_ref[0]
    out2_0 = dinv0 * _mcat(q0_ref) + (dinv0 * dinv0) * xw2_0 + b2
    out2_1 = dinv1 * _mcat(q1_ref) + (dinv0 * dinv1) * xw2_0 \
        + (dinv1 * dinv1) * xw2_ref[1] + b2
    out2_2 = dinv2 * _mcat(q2_ref) + (dinv0 * dinv2) * xw2_0 \
        + (dinv2 * dinv2) * xw2_ref[2] + b2
    out20_ref[...] = out2_0
    l1w = l1w_ref[...]
    l1b = l1b_ref[...]
    iota = lax.broadcasted_iota(jnp.int32, (RB, M), 1)
    h = jnp.zeros((RB, M), jnp.float32)
    for blk in (out2_0, out2_1, out2_2):
        alll = jnp.dot(jnp.maximum(blk, 0.0), l1w,
                       preferred_element_type=jnp.float32) + l1b
        mx = jnp.max(alll, axis=1, keepdims=True)
        cls = jnp.min(jnp.where(alll == mx, iota, M), axis=1, keepdims=True)
        h = h + (iota == cls).astype(jnp.float32)
    h_ref[...] = h


def _tc_fin1(macc2, degacc, xw2, b2, l1w, l1b):
    return pl.pallas_call(
        _fin1_body,
        grid=(GRID,),
        in_specs=[_mid((NC, RB, M), 0), _mid((NC, RB, M), GRID),
                  _mid((NC, RB, M), 2 * GRID),
                  _rows3((NC, RB, L)), _rows3((3, RB, D)),
                  _full((1, D)), _full((D, M)), _full((1, M))],
        out_specs=[_rows((RB, D)), _rows((RB, M))],
        out_shape=[jax.ShapeDtypeStruct((N, D), jnp.float32),
                   jax.ShapeDtypeStruct((N, M), jnp.float32)],
    )(macc2, macc2, macc2, degacc, xw2, b2, l1w, l1b)


def _fin2_body(h_ref, out20_ref, f_ref, hs_ref, hf_ref, dots_ref):
    h = h_ref[...]
    mask = (h > 0.0).astype(jnp.float32)
    hf = lax.dot_general(mask, out20_ref[...], (((0,), (0,)), ((), ())),
                         preferred_element_type=jnp.float32)
    hf_ref[...] = hf
    scale = float(D) ** (-0.5)
    d1 = jnp.dot(f_ref[...], hf.T, preferred_element_type=jnp.float32) * scale
    dots_ref[...] = jnp.concatenate([d1, d1, d1], axis=0)
    mx = jnp.max(h, axis=0, keepdims=True)
    eh = jnp.exp(h - mx)
    hs_ref[...] = eh / jnp.sum(eh, axis=0, keepdims=True)


def _tc_fin2(h, out20, f):
    return pl.pallas_call(
        _fin2_body,
        out_shape=[jax.ShapeDtypeStruct((N, M), jnp.float32),
                   jax.ShapeDtypeStruct((M, D), jnp.float32),
                   jax.ShapeDtypeStruct((3 * N, M), jnp.float32)],
    )(h, out20, f)


def kernel(edge_index, features, lin_W, lin_b, gcn1_W, gcn1_b, gcn2_W, gcn2_b,
           lin1_W, lin1_b):
    src = edge_index[0].astype(jnp.int32)
    dst = edge_index[1].astype(jnp.int32)
    pad = jnp.zeros((EP - E,), jnp.int32)
    srcp = jnp.concatenate([src, pad]).reshape(NROWS, 1, CH)
    dstp = jnp.concatenate([dst, pad]).reshape(NROWS, 1, CH)

    _sim_kernel = _make_sim_kernel()
    _agg_kernel = _make_agg_kernel()
    fhat, dstack, xw1 = _tc_prep(features, lin_W, lin_b, gcn1_W)
    idx0, idx1, idx2, degacc = _sim_kernel(dstack, fhat, srcp, dstp)
    degacc = degacc.reshape(NC, ND, L)
    (p1,) = _tc_p1(degacc, xw1)
    (macc1,) = _agg_kernel(p1.reshape(NC * N, M), srcp, idx0, idx1, idx2)
    macc1 = macc1.reshape(NC, ACC_ROWS, M)
    p2, xw2 = _tc_mid(macc1, degacc, xw1, gcn1_b.reshape(1, D), gcn2_W)
    (macc2,) = _agg_kernel(p2.reshape(NC * N, M), srcp, idx0, idx1, idx2)
    macc2 = macc2.reshape(NC, ACC_ROWS, M)
    out20, h = _tc_fin1(macc2, degacc, xw2, gcn2_b.reshape(1, D), lin1_W,
                        lin1_b.reshape(1, M))
    hs, hf, dots = _tc_fin2(h, out20, features)
    return (hs, hf, dots)


# speedup vs baseline: 9.8759x; 1.3307x over previous
import functools

import jax
import jax.numpy as jnp
from jax import lax
from jax.experimental import pallas as pl
from jax.experimental.pallas import tpu as pltpu
from jax.experimental.pallas import tpu_sc as plsc

N = 10000
D = 128
T = 2
M = 64
E = 160000
L = 16
NC = 2
NS = 16
CH = 128
NROWS = 1280
EP = NROWS * CH
TRASH = 3 * N
ACC_ROWS = 30080
ACC_PT = ACC_ROWS // NS
ND = 10240
DEG_PT = ND // NS
RB = 2000
GRID = N // RB

_sc_params = pltpu.CompilerParams(use_tc_tiling_on_sc=False,
                                  needs_layout_passes=False)


def _sim_body(dstack, fhat, srcp, dstp,
              idx012, degacc,
              sbufA, dbufA, sbufB, dbufB, drowsA, frowsA, drowsB, frowsB,
              i012b, valb, zdeg, dacc,
              semAd, semAf, semBd, semBf):
    c = lax.axis_index("c")
    s = lax.axis_index("s")
    wid = s * NC + c
    iota = lax.iota(jnp.int32, L)
    zf = jnp.zeros((L,), jnp.float32)
    zi = jnp.zeros((L,), jnp.int32)
    one = jnp.ones((L,), jnp.float32)

    def zr(r, _):
        zdeg[r, :] = zf
        return 0
    lax.fori_loop(0, DEG_PT, zr, 0)
    pltpu.sync_copy(zdeg, dacc.at[pl.ds(s * DEG_PT, DEG_PT)])
    plsc.subcore_barrier()

    rows_pt = NROWS // (NC * NS)
    base = wid * rows_pt

    def stage(row, sbuf, dbuf, drows, frows, semd, semf):
        pltpu.sync_copy(srcp.at[row, 0], sbuf)
        pltpu.sync_copy(dstp.at[row, 0], dbuf)
        pltpu.async_copy(dstack.at[sbuf], drows, semd)
        pltpu.async_copy(fhat.at[dbuf], frows, semf)

    def compute(row, dbuf, drows, frows):
        for g in range(CH // L):
            rowv = g * L + iota
            dstv = dbuf[pl.ds(g * L, L)]

            def dot16(q, car):
                a1, a2, colv = car
                for _ in range(16):
                    dv1 = plsc.load_gather(drows, [rowv, colv])
                    dv2 = plsc.load_gather(drows, [rowv, colv + D])
                    fv = plsc.load_gather(frows, [rowv, colv])
                    a1 = a1 + dv1 * fv
                    a2 = a2 + dv2 * fv
                    colv = colv + 1
                return (a1, a2, colv)

            acc1, acc2, _ = lax.fori_loop(0, D // 16, dot16, (zf, zf, zi))
            eid = row * CH + g * L + iota
            valid = eid < E
            w1 = valid & (acc1 > 0.0)
            w2 = valid & (acc2 > 0.0)
            h, cc = g // 4, (g % 4) * L
            i012b[h, 0, pl.ds(cc, L)] = jnp.where(valid, dstv, TRASH)
            i012b[h, 1, pl.ds(cc, L)] = jnp.where(w1, dstv + N, TRASH)
            i012b[h, 2, pl.ds(cc, L)] = jnp.where(w2, dstv + 2 * N, TRASH)
            plsc.store_scatter(valb, [rowv, zi], jnp.where(valid, one, 0.0))
            plsc.store_scatter(valb, [rowv, zi + 1], jnp.where(w1, one, 0.0))
            plsc.store_scatter(valb, [rowv, zi + 2], jnp.where(w2, one, 0.0))
        pltpu.sync_copy(valb, dacc.at[dbuf], add=True)
        pltpu.sync_copy(i012b, idx012.at[row])

    stage(base, sbufA, dbufA, drowsA, frowsA, semAd, semAf)

    def pair(p, _):
        rowA = base + 2 * p
        rowB = rowA + 1
        stage(rowB, sbufB, dbufB, drowsB, frowsB, semBd, semBf)
        pltpu.make_async_copy(dstack.at[sbufA], drowsA, semAd).wait()
        pltpu.make_async_copy(fhat.at[dbufA], frowsA, semAf).wait()
        compute(rowA, dbufA, drowsA, frowsA)
        stage((rowA + 2) % NROWS, sbufA, dbufA, drowsA, frowsA, semAd, semAf)
        pltpu.make_async_copy(dstack.at[sbufB], drowsB, semBd).wait()
        pltpu.make_async_copy(fhat.at[dbufB], frowsB, semBf).wait()
        compute(rowB, dbufB, drowsB, frowsB)
        return 0

    lax.fori_loop(0, rows_pt // 2, pair, 0)
    pltpu.make_async_copy(dstack.at[sbufA], drowsA, semAd).wait()
    pltpu.make_async_copy(fhat.at[dbufA], frowsA, semAf).wait()
    plsc.subcore_barrier()
    pltpu.sync_copy(dacc.at[pl.ds(s * DEG_PT, DEG_PT)],
                    degacc.at[pl.ds(c * ND + s * DEG_PT, DEG_PT)])


def _make_sim_kernel():
  return functools.partial(
    pl.kernel,
    out_type=[
        jax.ShapeDtypeStruct((NROWS, 2, 3, CH // 2), jnp.int32),
        jax.ShapeDtypeStruct((NC * ND, L), jnp.float32),
    ],
    mesh=plsc.VectorSubcoreMesh(core_axis_name="c", subcore_axis_name="s",
                                num_cores=NC, num_subcores=NS),
    scratch_types=[
        pltpu.VMEM((CH,), jnp.int32),
        pltpu.VMEM((CH,), jnp.int32),
        pltpu.VMEM((CH,), jnp.int32),
        pltpu.VMEM((CH,), jnp.int32),
        pltpu.VMEM((CH, 2 * D), jnp.float32),
        pltpu.VMEM((CH, D), jnp.float32),
        pltpu.VMEM((CH, 2 * D), jnp.float32),
        pltpu.VMEM((CH, D), jnp.float32),
        pltpu.VMEM((2, 3, CH // 2), jnp.int32),
        pltpu.VMEM((CH, L), jnp.float32),
        pltpu.VMEM((DEG_PT, L), jnp.float32),
        pltpu.VMEM_SHARED((ND, L), jnp.float32),
        pltpu.SemaphoreType.DMA,
        pltpu.SemaphoreType.DMA,
        pltpu.SemaphoreType.DMA,
        pltpu.SemaphoreType.DMA,
    ],
    compiler_params=_sc_params,
  )(_sim_body)


def _agg_body(table, srcp, idx012,
              macc,
              sbufA, ibufA, sbufB, ibufB, rowsA, rowsB, acc, semA, semB):
    c = lax.axis_index("c")
    s = lax.axis_index("s")
    zf = jnp.zeros((L,), jnp.float32)
    CB = CH // 2

    def zr2(r, _):
        for q in range(M // L):
            rowsA[r, pl.ds(q * L, L)] = zf
            rowsB[r, pl.ds(q * L, L)] = zf
        return 0
    lax.fori_loop(0, CB, zr2, 0)
    nfull = ACC_PT // CB
    rem = ACC_PT - nfull * CB
    for q in range(nfull):
        pltpu.sync_copy(rowsA, acc.at[pl.ds(s * ACC_PT + q * CB, CB)])
    pltpu.sync_copy(rowsA.at[pl.ds(0, rem)],
                    acc.at[pl.ds(s * ACC_PT + nfull * CB, rem)])
    plsc.subcore_barrier()

    nch = 2 * NROWS // NS
    base = s * nch
    coff = c * N

    def stage(ch64, sbuf, ibuf, rows, sem):
        row = ch64 // 2
        h = ch64 % 2
        pltpu.sync_copy(srcp.at[row, h], sbuf)
        pltpu.sync_copy(idx012.at[row, h], ibuf)
        for g in range(CB // L):
            sbuf[pl.ds(g * L, L)] = sbuf[pl.ds(g * L, L)] + coff
        pltpu.async_copy(table.at[sbuf], rows, sem)

    def scatter(rows, ibuf):
        pltpu.sync_copy(rows, acc.at[ibuf.at[0]], add=True)
        pltpu.sync_copy(rows, acc.at[ibuf.at[1]], add=True)
        pltpu.sync_copy(rows, acc.at[ibuf.at[2]], add=True)

    stage(base, sbufA, ibufA, rowsA, semA)

    def pair(p, _):
        cA = base + 2 * p
        cB = cA + 1
        stage(cB, sbufB, ibufB, rowsB, semB)
        pltpu.make_async_copy(table.at[sbufA], rowsA, semA).wait()
        scatter(rowsA, ibufA)
        stage((cA + 2) % (2 * NROWS), sbufA, ibufA, rowsA, semA)
        pltpu.make_async_copy(table.at[sbufB], rowsB, semB).wait()
        scatter(rowsB, ibufB)
        return 0

    lax.fori_loop(0, nch // 2, pair, 0)
    pltpu.make_async_copy(table.at[sbufA], rowsA, semA).wait()
    plsc.subcore_barrier()
    pltpu.sync_copy(acc.at[pl.ds(s * ACC_PT, ACC_PT)],
                    macc.at[pl.ds(c * ACC_ROWS + s * ACC_PT, ACC_PT)])


def _make_agg_kernel():
  return functools.partial(
    pl.kernel,
    out_type=[jax.ShapeDtypeStruct((NC * ACC_ROWS, M), jnp.float32)],
    mesh=plsc.VectorSubcoreMesh(core_axis_name="c", subcore_axis_name="s",
                                num_cores=NC, num_subcores=NS),
    scratch_types=[
        pltpu.VMEM((CH // 2,), jnp.int32),
        pltpu.VMEM((3, CH // 2), jnp.int32),
        pltpu.VMEM((CH // 2,), jnp.int32),
        pltpu.VMEM((3, CH // 2), jnp.int32),
        pltpu.VMEM((CH // 2, M), jnp.float32),
        pltpu.VMEM((CH // 2, M), jnp.float32),
        pltpu.VMEM_SHARED((ACC_ROWS, M), jnp.float32),
        pltpu.SemaphoreType.DMA,
        pltpu.SemaphoreType.DMA,
    ],
    compiler_params=_sc_params,
  )(_agg_body)


def _full(shape):
    return pl.BlockSpec(shape, lambda i: tuple(0 for _ in shape))


def _rows(shape):
    return pl.BlockSpec(shape, lambda i: (i,) + tuple(0 for _ in shape[1:]))


def _mid(shape, off):
    return pl.BlockSpec(shape, lambda i, _o=off: (0, _o + i, 0))


def _rows3(shape):
    return pl.BlockSpec(shape, lambda i: (0, i, 0))


def _prep_body(f_ref, linw_ref, linb_ref, w1_ref,
               fhat_ref, dstack_ref, xw1_ref):
    f = f_ref[...]
    nf = jnp.maximum(jnp.sqrt(jnp.sum(f * f, axis=1, keepdims=True)), 1e-8)
    fhat = f / nf
    fhat_ref[...] = fhat
    for i in range(T):
        ti = jnp.dot(f, linw_ref[i], preferred_element_type=jnp.float32) \
            + linb_ref[i:i + 1, :]
        nt = jnp.maximum(jnp.sqrt(jnp.sum(ti * ti, axis=1, keepdims=True)), 1e-8)
        dstack_ref[:, i * D:(i + 1) * D] = ti / nt - fhat
    xw1_ref[...] = jnp.dot(jnp.maximum(f, 0.0), w1_ref[...],
                           preferred_element_type=jnp.float32)


def _tc_prep(f, lin_W, lin_b, gcn1_W):
    return pl.pallas_call(
        _prep_body,
        grid=(GRID,),
        in_specs=[_rows((RB, D)), _full((T, D, D)), _full((T, D)),
                  _full((D, D))],
        out_specs=[_rows((RB, D)), _rows((RB, 2 * D)), _rows((RB, D))],
        out_shape=[jax.ShapeDtypeStruct((N, D), jnp.float32),
                   jax.ShapeDtypeStruct((N, 2 * D), jnp.float32),
                   jax.ShapeDtypeStruct((N, D), jnp.float32)],
    )(f, lin_W, lin_b, gcn1_W)


def _dinvs(degacc):
    dsum = degacc[0] + degacc[1]
    indeg = dsum[:, 0:1]
    dinv0 = lax.rsqrt(indeg + 1.0)
    dinv1 = lax.rsqrt(dsum[:, 1:2] + 2.0)
    dinv2 = lax.rsqrt(dsum[:, 2:3] + 2.0)
    return dinv0, dinv1, dinv2


def _p1_body(degacc_ref, xw1_ref, p1_ref):
    dinv0, _, _ = _dinvs(degacc_ref[...])
    p1 = dinv0 * xw1_ref[...]
    p1_ref[0] = p1[:, :M]
    p1_ref[1] = p1[:, M:]


def _tc_p1(degacc, xw1):
    return pl.pallas_call(
        _p1_body,
        grid=(GRID,),
        in_specs=[_rows3((NC, RB, L)), _rows((RB, D))],
        out_specs=[_rows3((NC, RB, M))],
        out_shape=[jax.ShapeDtypeStruct((NC, N, M), jnp.float32)],
    )(degacc, xw1)


def _mcat(m_ref):
    return jnp.concatenate([m_ref[0], m_ref[1]], axis=1)


def _mid_body(m0_ref, m1_ref, m2_ref, degacc_ref, xw1_ref,
              b1_ref, w2_ref, p2_ref, xw2_ref):
    dinv0, dinv1, dinv2 = _dinvs(degacc_ref[...])
    xw1 = xw1_ref[...]
    b1 = b1_ref[...]
    w2 = w2_ref[...]
    out1_0 = dinv0 * _mcat(m0_ref) + (dinv0 * dinv0) * xw1 + b1
    out1_1 = dinv1 * _mcat(m1_ref) + (dinv1 * (dinv0 + dinv1)) * xw1 + b1
    out1_2 = dinv2 * _mcat(m2_ref) + (dinv2 * (dinv0 + dinv2)) * xw1 + b1
    xw2_0 = jnp.dot(jnp.maximum(out1_0, 0.0), w2,
                    preferred_element_type=jnp.float32)
    xw2_ref[0] = xw2_0
    xw2_ref[1] = jnp.dot(jnp.maximum(out1_1, 0.0), w2,
                         preferred_element_type=jnp.float32)
    xw2_ref[2] = jnp.dot(jnp.maximum(out1_2, 0.0), w2,
                         preferred_element_type=jnp.float32)
    p2 = dinv0 * xw2_0
    p2_ref[0] = p2[:, :M]
    p2_ref[1] = p2[:, M:]


def _tc_mid(macc1, degacc, xw1, b1, w2):
    return pl.pallas_call(
        _mid_body,
        grid=(GRID,),
        in_specs=[_mid((NC, RB, M), 0), _mid((NC, RB, M), GRID),
                  _mid((NC, RB, M), 2 * GRID),
                  _rows3((NC, RB, L)), _rows((RB, D)),
                  _full((1, D)), _full((D, D))],
        out_specs=[_rows3((NC, RB, M)), _rows3((3, RB, D))],
        out_shape=[jax.ShapeDtypeStruct((NC, N, M), jnp.float32),
                   jax.ShapeDtypeStruct((3, N, D), jnp.float32)],
    )(macc1, macc1, macc1, degacc, xw1, b1, w2)


def _fin1_body(q0_ref, q1_ref, q2_ref, degacc_ref, xw2_ref,
               b2_ref, l1w_ref, l1b_ref, out20_ref, h_ref):
    dinv0, dinv1, dinv2 = _dinvs(degacc_ref[...])
    b2 = b2_ref[...]
    xw2_0 = xw2_ref[0]
    out2_0 = dinv0 * _mcat(q0_ref) + (dinv0 * dinv0) * xw2_0 + b2
    out2_1 = dinv1 * _mcat(q1_ref) + (dinv0 * dinv1) * xw2_0 \
        + (dinv1 * dinv1) * xw2_ref[1] + b2
    out2_2 = dinv2 * _mcat(q2_ref) + (dinv0 * dinv2) * xw2_0 \
        + (dinv2 * dinv2) * xw2_ref[2] + b2
    out20_ref[...] = out2_0
    l1w = l1w_ref[...]
    l1b = l1b_ref[...]
    iota = lax.broadcasted_iota(jnp.int32, (RB, M), 1)
    h = jnp.zeros((RB, M), jnp.float32)
    for blk in (out2_0, out2_1, out2_2):
        alll = jnp.dot(jnp.maximum(blk, 0.0), l1w,
                       preferred_element_type=jnp.float32) + l1b
        mx = jnp.max(alll, axis=1, keepdims=True)
        cls = jnp.min(jnp.where(alll == mx, iota, M), axis=1, keepdims=True)
        h = h + (iota == cls).astype(jnp.float32)
    h_ref[...] = h


def _tc_fin1(macc2, degacc, xw2, b2, l1w, l1b):
    return pl.pallas_call(
        _fin1_body,
        grid=(GRID,),
        in_specs=[_mid((NC, RB, M), 0), _mid((NC, RB, M), GRID),
                  _mid((NC, RB, M), 2 * GRID),
                  _rows3((NC, RB, L)), _rows3((3, RB, D)),
                  _full((1, D)), _full((D, M)), _full((1, M))],
        out_specs=[_rows((RB, D)), _rows((RB, M))],
        out_shape=[jax.ShapeDtypeStruct((N, D), jnp.float32),
                   jax.ShapeDtypeStruct((N, M), jnp.float32)],
    )(macc2, macc2, macc2, degacc, xw2, b2, l1w, l1b)


def _fin2_body(h_ref, out20_ref, f_ref, hs_ref, hf_ref, dots_ref):
    h = h_ref[...]
    mask = (h > 0.0).astype(jnp.float32)
    hf = lax.dot_general(mask, out20_ref[...], (((0,), (0,)), ((), ())),
                         preferred_element_type=jnp.float32)
    hf_ref[...] = hf
    scale = float(D) ** (-0.5)
    d1 = jnp.dot(f_ref[...], hf.T, preferred_element_type=jnp.float32) * scale
    dots_ref[...] = jnp.concatenate([d1, d1, d1], axis=0)
    mx = jnp.max(h, axis=0, keepdims=True)
    eh = jnp.exp(h - mx)
    hs_ref[...] = eh / jnp.sum(eh, axis=0, keepdims=True)


def _tc_fin2(h, out20, f):
    return pl.pallas_call(
        _fin2_body,
        out_shape=[jax.ShapeDtypeStruct((N, M), jnp.float32),
                   jax.ShapeDtypeStruct((M, D), jnp.float32),
                   jax.ShapeDtypeStruct((3 * N, M), jnp.float32)],
    )(h, out20, f)


def kernel(edge_index, features, lin_W, lin_b, gcn1_W, gcn1_b, gcn2_W, gcn2_b,
           lin1_W, lin1_b):
    src = edge_index[0].astype(jnp.int32)
    dst = edge_index[1].astype(jnp.int32)
    pad = jnp.zeros((EP - E,), jnp.int32)
    srcp = jnp.concatenate([src, pad]).reshape(NROWS, 1, CH)
    dstp = jnp.concatenate([dst, pad]).reshape(NROWS, 1, CH)
    srcp64 = srcp.reshape(NROWS, 2, CH // 2)

    _sim_kernel = _make_sim_kernel()
    _agg_kernel = _make_agg_kernel()
    fhat, dstack, xw1 = _tc_prep(features, lin_W, lin_b, gcn1_W)
    idx012, degacc = _sim_kernel(dstack, fhat, srcp, dstp)
    degacc = degacc.reshape(NC, ND, L)
    (p1,) = _tc_p1(degacc, xw1)
    (macc1,) = _agg_kernel(p1.reshape(NC * N, M), srcp64, idx012)
    macc1 = macc1.reshape(NC, ACC_ROWS, M)
    p2, xw2 = _tc_mid(macc1, degacc, xw1, gcn1_b.reshape(1, D), gcn2_W)
    (macc2,) = _agg_kernel(p2.reshape(NC * N, M), srcp64, idx012)
    macc2 = macc2.reshape(NC, ACC_ROWS, M)
    out20, h = _tc_fin1(macc2, degacc, xw2, gcn2_b.reshape(1, D), lin1_W,
                        lin1_b.reshape(1, M))
    hs, hf, dots = _tc_fin2(h, out20, features)
    return (hs, hf, dots)


# speedup vs baseline: 15.7493x; 1.5947x over previous
import functools

import jax
import jax.numpy as jnp
from jax import lax
from jax.experimental import pallas as pl
from jax.experimental.pallas import tpu as pltpu
from jax.experimental.pallas import tpu_sc as plsc

N = 10000
D = 128
T = 2
M = 64
E = 160000
L = 16
NC = 2
NS = 16
CH = 128
NROWS = 1280
EP = NROWS * CH
TRASH = 3 * N
ACC_ROWS = 30080
ACC_PT = ACC_ROWS // NS
ND = 10240
DEG_PT = ND // NS
RB = 2000
GRID = N // RB

_sc_params = pltpu.CompilerParams(use_tc_tiling_on_sc=False,
                                  needs_layout_passes=False)


def _sim_body(dstack, fhat, srcp, dstp,
              idx012, degacc,
              sbufA, dbufA, sbufB, dbufB, drowsA, frowsA, drowsB, frowsB,
              i012b, valb, accb, zdeg, dacc,
              semAd, semAf, semBd, semBf):
    c = lax.axis_index("c")
    s = lax.axis_index("s")
    wid = s * NC + c
    iota = lax.iota(jnp.int32, L)
    zf = jnp.zeros((L,), jnp.float32)
    zi = jnp.zeros((L,), jnp.int32)
    one = jnp.ones((L,), jnp.float32)

    def zr(r, _):
        zdeg[r, :] = zf
        return 0
    lax.fori_loop(0, DEG_PT, zr, 0)
    pltpu.sync_copy(zdeg, dacc.at[pl.ds(s * DEG_PT, DEG_PT)])
    plsc.subcore_barrier()

    rows_pt = NROWS // (NC * NS)
    base = wid * rows_pt

    def stage(row, sbuf, dbuf, drows, frows, semd, semf):
        pltpu.sync_copy(srcp.at[row, 0], sbuf)
        pltpu.sync_copy(dstp.at[row, 0], dbuf)
        pltpu.async_copy(dstack.at[sbuf], drows, semd)
        pltpu.async_copy(fhat.at[dbuf], frows, semf)

    def compute(row, dbuf, drows, frows):
        lastlane = iota == (L - 1)

        def edge(kk, _):
            a1 = zf
            a2 = zf
            for q in range(D // L):
                fv = frows[kk, pl.ds(q * L, L)]
                a1 = a1 + drows[kk, pl.ds(q * L, L)] * fv
                a2 = a2 + drows[kk, pl.ds(D + q * L, L)] * fv
            kv = jnp.full((L,), kk, jnp.int32)
            plsc.store_scatter(accb, [zi, kv], plsc.cumsum(a1), mask=lastlane)
            plsc.store_scatter(accb, [zi + 1, kv], plsc.cumsum(a2), mask=lastlane)
            return 0

        lax.fori_loop(0, CH, edge, 0)
        for g in range(CH // L):
            rowv = g * L + iota
            dstv = dbuf[pl.ds(g * L, L)]
            acc1 = accb[0, pl.ds(g * L, L)]
            acc2 = accb[1, pl.ds(g * L, L)]
            eid = row * CH + g * L + iota
            valid = eid < E
            w1 = valid & (acc1 > 0.0)
            w2 = valid & (acc2 > 0.0)
            h, cc = g // 4, (g % 4) * L
            i012b[h, 0, pl.ds(cc, L)] = jnp.where(valid, dstv, TRASH)
            i012b[h, 1, pl.ds(cc, L)] = jnp.where(w1, dstv + N, TRASH)
            i012b[h, 2, pl.ds(cc, L)] = jnp.where(w2, dstv + 2 * N, TRASH)
            plsc.store_scatter(valb, [rowv, zi], jnp.where(valid, one, 0.0))
            plsc.store_scatter(valb, [rowv, zi + 1], jnp.where(w1, one, 0.0))
            plsc.store_scatter(valb, [rowv, zi + 2], jnp.where(w2, one, 0.0))
        pltpu.sync_copy(valb, dacc.at[dbuf], add=True)
        pltpu.sync_copy(i012b, idx012.at[row])

    stage(base, sbufA, dbufA, drowsA, frowsA, semAd, semAf)

    def pair(p, _):
        rowA = base + 2 * p
        rowB = rowA + 1
        stage(rowB, sbufB, dbufB, drowsB, frowsB, semBd, semBf)
        pltpu.make_async_copy(dstack.at[sbufA], drowsA, semAd).wait()
        pltpu.make_async_copy(fhat.at[dbufA], frowsA, semAf).wait()
        compute(rowA, dbufA, drowsA, frowsA)
        stage((rowA + 2) % NROWS, sbufA, dbufA, drowsA, frowsA, semAd, semAf)
        pltpu.make_async_copy(dstack.at[sbufB], drowsB, semBd).wait()
        pltpu.make_async_copy(fhat.at[dbufB], frowsB, semBf).wait()
        compute(rowB, dbufB, drowsB, frowsB)
        return 0

    lax.fori_loop(0, rows_pt // 2, pair, 0)
    pltpu.make_async_copy(dstack.at[sbufA], drowsA, semAd).wait()
    pltpu.make_async_copy(fhat.at[dbufA], frowsA, semAf).wait()
    plsc.subcore_barrier()
    pltpu.sync_copy(dacc.at[pl.ds(s * DEG_PT, DEG_PT)],
                    degacc.at[pl.ds(c * ND + s * DEG_PT, DEG_PT)])


def _make_sim_kernel():
  return functools.partial(
    pl.kernel,
    out_type=[
        jax.ShapeDtypeStruct((NROWS, 2, 3, CH // 2), jnp.int32),
        jax.ShapeDtypeStruct((NC * ND, L), jnp.float32),
    ],
    mesh=plsc.VectorSubcoreMesh(core_axis_name="c", subcore_axis_name="s",
                                num_cores=NC, num_subcores=NS),
    scratch_types=[
        pltpu.VMEM((CH,), jnp.int32),
        pltpu.VMEM((CH,), jnp.int32),
        pltpu.VMEM((CH,), jnp.int32),
        pltpu.VMEM((CH,), jnp.int32),
        pltpu.VMEM((CH, 2 * D), jnp.float32),
        pltpu.VMEM((CH, D), jnp.float32),
        pltpu.VMEM((CH, 2 * D), jnp.float32),
        pltpu.VMEM((CH, D), jnp.float32),
        pltpu.VMEM((2, 3, CH // 2), jnp.int32),
        pltpu.VMEM((CH, L), jnp.float32),
        pltpu.VMEM((2, CH), jnp.float32),
        pltpu.VMEM((DEG_PT, L), jnp.float32),
        pltpu.VMEM_SHARED((ND, L), jnp.float32),
        pltpu.SemaphoreType.DMA,
        pltpu.SemaphoreType.DMA,
        pltpu.SemaphoreType.DMA,
        pltpu.SemaphoreType.DMA,
    ],
    compiler_params=_sc_params,
  )(_sim_body)


def _agg_body(table, srcp, idx012,
              macc,
              sbufA, ibufA, sbufB, ibufB, rowsA, rowsB, acc, semA, semB):
    c = lax.axis_index("c")
    s = lax.axis_index("s")
    zf = jnp.zeros((L,), jnp.float32)
    CB = CH // 2

    def zr2(r, _):
        for q in range(M // L):
            rowsA[r, pl.ds(q * L, L)] = zf
            rowsB[r, pl.ds(q * L, L)] = zf
        return 0
    lax.fori_loop(0, CB, zr2, 0)
    nfull = ACC_PT // CB
    rem = ACC_PT - nfull * CB
    for q in range(nfull):
        pltpu.sync_copy(rowsA, acc.at[pl.ds(s * ACC_PT + q * CB, CB)])
    pltpu.sync_copy(rowsA.at[pl.ds(0, rem)],
                    acc.at[pl.ds(s * ACC_PT + nfull * CB, rem)])
    plsc.subcore_barrier()

    nch = 2 * NROWS // NS
    base = s * nch
    coff = c * N

    def stage(ch64, sbuf, ibuf, rows, sem):
        row = ch64 // 2
        h = ch64 % 2
        pltpu.sync_copy(srcp.at[row, h], sbuf)
        pltpu.sync_copy(idx012.at[row, h], ibuf)
        for g in range(CB // L):
            sbuf[pl.ds(g * L, L)] = sbuf[pl.ds(g * L, L)] + coff
        pltpu.async_copy(table.at[sbuf], rows, sem)

    def scatter(rows, ibuf):
        pltpu.sync_copy(rows, acc.at[ibuf.at[0]], add=True)
        pltpu.sync_copy(rows, acc.at[ibuf.at[1]], add=True)
        pltpu.sync_copy(rows, acc.at[ibuf.at[2]], add=True)

    stage(base, sbufA, ibufA, rowsA, semA)

    def pair(p, _):
        cA = base + 2 * p
        cB = cA + 1
        stage(cB, sbufB, ibufB, rowsB, semB)
        pltpu.make_async_copy(table.at[sbufA], rowsA, semA).wait()
        scatter(rowsA, ibufA)
        stage((cA + 2) % (2 * NROWS), sbufA, ibufA, rowsA, semA)
        pltpu.make_async_copy(table.at[sbufB], rowsB, semB).wait()
        scatter(rowsB, ibufB)
        return 0

    lax.fori_loop(0, nch // 2, pair, 0)
    pltpu.make_async_copy(table.at[sbufA], rowsA, semA).wait()
    plsc.subcore_barrier()
    pltpu.sync_copy(acc.at[pl.ds(s * ACC_PT, ACC_PT)],
                    macc.at[pl.ds(c * ACC_ROWS + s * ACC_PT, ACC_PT)])


def _make_agg_kernel():
  return functools.partial(
    pl.kernel,
    out_type=[jax.ShapeDtypeStruct((NC * ACC_ROWS, M), jnp.float32)],
    mesh=plsc.VectorSubcoreMesh(core_axis_name="c", subcore_axis_name="s",
                                num_cores=NC, num_subcores=NS),
    scratch_types=[
        pltpu.VMEM((CH // 2,), jnp.int32),
        pltpu.VMEM((3, CH // 2), jnp.int32),
        pltpu.VMEM((CH // 2,), jnp.int32),
        pltpu.VMEM((3, CH // 2), jnp.int32),
        pltpu.VMEM((CH // 2, M), jnp.float32),
        pltpu.VMEM((CH // 2, M), jnp.float32),
        pltpu.VMEM_SHARED((ACC_ROWS, M), jnp.float32),
        pltpu.SemaphoreType.DMA,
        pltpu.SemaphoreType.DMA,
    ],
    compiler_params=_sc_params,
  )(_agg_body)


def _full(shape):
    return pl.BlockSpec(shape, lambda i: tuple(0 for _ in shape))


def _rows(shape):
    return pl.BlockSpec(shape, lambda i: (i,) + tuple(0 for _ in shape[1:]))


def _mid(shape, off):
    return pl.BlockSpec(shape, lambda i, _o=off: (0, _o + i, 0))


def _rows3(shape):
    return pl.BlockSpec(shape, lambda i: (0, i, 0))


def _prep_body(f_ref, linw_ref, linb_ref, w1_ref,
               fhat_ref, dstack_ref, xw1_ref):
    f = f_ref[...]
    nf = jnp.maximum(jnp.sqrt(jnp.sum(f * f, axis=1, keepdims=True)), 1e-8)
    fhat = f / nf
    fhat_ref[...] = fhat
    for i in range(T):
        ti = jnp.dot(f, linw_ref[i], preferred_element_type=jnp.float32) \
            + linb_ref[i:i + 1, :]
        nt = jnp.maximum(jnp.sqrt(jnp.sum(ti * ti, axis=1, keepdims=True)), 1e-8)
        dstack_ref[:, i * D:(i + 1) * D] = ti / nt - fhat
    xw1_ref[...] = jnp.dot(jnp.maximum(f, 0.0), w1_ref[...],
                           preferred_element_type=jnp.float32)


def _tc_prep(f, lin_W, lin_b, gcn1_W):
    return pl.pallas_call(
        _prep_body,
        grid=(GRID,),
        in_specs=[_rows((RB, D)), _full((T, D, D)), _full((T, D)),
                  _full((D, D))],
        out_specs=[_rows((RB, D)), _rows((RB, 2 * D)), _rows((RB, D))],
        out_shape=[jax.ShapeDtypeStruct((N, D), jnp.float32),
                   jax.ShapeDtypeStruct((N, 2 * D), jnp.float32),
                   jax.ShapeDtypeStruct((N, D), jnp.float32)],
    )(f, lin_W, lin_b, gcn1_W)


def _dinvs(degacc):
    dsum = degacc[0] + degacc[1]
    indeg = dsum[:, 0:1]
    dinv0 = lax.rsqrt(indeg + 1.0)
    dinv1 = lax.rsqrt(dsum[:, 1:2] + 2.0)
    dinv2 = lax.rsqrt(dsum[:, 2:3] + 2.0)
    return dinv0, dinv1, dinv2


def _p1_body(degacc_ref, xw1_ref, p1_ref):
    dinv0, _, _ = _dinvs(degacc_ref[...])
    p1 = dinv0 * xw1_ref[...]
    p1_ref[0] = p1[:, :M]
    p1_ref[1] = p1[:, M:]


def _tc_p1(degacc, xw1):
    return pl.pallas_call(
        _p1_body,
        grid=(GRID,),
        in_specs=[_rows3((NC, RB, L)), _rows((RB, D))],
        out_specs=[_rows3((NC, RB, M))],
        out_shape=[jax.ShapeDtypeStruct((NC, N, M), jnp.float32)],
    )(degacc, xw1)


def _mcat(m_ref):
    return jnp.concatenate([m_ref[0], m_ref[1]], axis=1)


def _mid_body(m0_ref, m1_ref, m2_ref, degacc_ref, xw1_ref,
              b1_ref, w2_ref, p2_ref, xw2_ref):
    dinv0, dinv1, dinv2 = _dinvs(degacc_ref[...])
    xw1 = xw1_ref[...]
    b1 = b1_ref[...]
    w2 = w2_ref[...]
    out1_0 = dinv0 * _mcat(m0_ref) + (dinv0 * dinv0) * xw1 + b1
    out1_1 = dinv1 * _mcat(m1_ref) + (dinv1 * (dinv0 + dinv1)) * xw1 + b1
    out1_2 = dinv2 * _mcat(m2_ref) + (dinv2 * (dinv0 + dinv2)) * xw1 + b1
    xw2_0 = jnp.dot(jnp.maximum(out1_0, 0.0), w2,
                    preferred_element_type=jnp.float32)
    xw2_ref[0] = xw2_0
    xw2_ref[1] = jnp.dot(jnp.maximum(out1_1, 0.0), w2,
                         preferred_element_type=jnp.float32)
    xw2_ref[2] = jnp.dot(jnp.maximum(out1_2, 0.0), w2,
                         preferred_element_type=jnp.float32)
    p2 = dinv0 * xw2_0
    p2_ref[0] = p2[:, :M]
    p2_ref[1] = p2[:, M:]


def _tc_mid(macc1, degacc, xw1, b1, w2):
    return pl.pallas_call(
        _mid_body,
        grid=(GRID,),
        in_specs=[_mid((NC, RB, M), 0), _mid((NC, RB, M), GRID),
                  _mid((NC, RB, M), 2 * GRID),
                  _rows3((NC, RB, L)), _rows((RB, D)),
                  _full((1, D)), _full((D, D))],
        out_specs=[_rows3((NC, RB, M)), _rows3((3, RB, D))],
        out_shape=[jax.ShapeDtypeStruct((NC, N, M), jnp.float32),
                   jax.ShapeDtypeStruct((3, N, D), jnp.float32)],
    )(macc1, macc1, macc1, degacc, xw1, b1, w2)


def _fin1_body(q0_ref, q1_ref, q2_ref, degacc_ref, xw2_ref,
               b2_ref, l1w_ref, l1b_ref, out20_ref, h_ref):
    dinv0, dinv1, dinv2 = _dinvs(degacc_ref[...])
    b2 = b2_ref[...]
    xw2_0 = xw2_ref[0]
    out2_0 = dinv0 * _mcat(q0_ref) + (dinv0 * dinv0) * xw2_0 + b2
    out2_1 = dinv1 * _mcat(q1_ref) + (dinv0 * dinv1) * xw2_0 \
        + (dinv1 * dinv1) * xw2_ref[1] + b2
    out2_2 = dinv2 * _mcat(q2_ref) + (dinv0 * dinv2) * xw2_0 \
        + (dinv2 * dinv2) * xw2_ref[2] + b2
    out20_ref[...] = out2_0
    l1w = l1w_ref[...]
    l1b = l1b_ref[...]
    iota = lax.broadcasted_iota(jnp.int32, (RB, M), 1)
    h = jnp.zeros((RB, M), jnp.float32)
    for blk in (out2_0, out2_1, out2_2):
        alll = jnp.dot(jnp.maximum(blk, 0.0), l1w,
                       preferred_element_type=jnp.float32) + l1b
        mx = jnp.max(alll, axis=1, keepdims=True)
        cls = jnp.min(jnp.where(alll == mx, iota, M), axis=1, keepdims=True)
        h = h + (iota == cls).astype(jnp.float32)
    h_ref[...] = h


def _tc_fin1(macc2, degacc, xw2, b2, l1w, l1b):
    return pl.pallas_call(
        _fin1_body,
        grid=(GRID,),
        in_specs=[_mid((NC, RB, M), 0), _mid((NC, RB, M), GRID),
                  _mid((NC, RB, M), 2 * GRID),
                  _rows3((NC, RB, L)), _rows3((3, RB, D)),
                  _full((1, D)), _full((D, M)), _full((1, M))],
        out_specs=[_rows((RB, D)), _rows((RB, M))],
        out_shape=[jax.ShapeDtypeStruct((N, D), jnp.float32),
                   jax.ShapeDtypeStruct((N, M), jnp.float32)],
    )(macc2, macc2, macc2, degacc, xw2, b2, l1w, l1b)


def _fin2_body(h_ref, out20_ref, f_ref, hs_ref, hf_ref, dots_ref):
    h = h_ref[...]
    mask = (h > 0.0).astype(jnp.float32)
    hf = lax.dot_general(mask, out20_ref[...], (((0,), (0,)), ((), ())),
                         preferred_element_type=jnp.float32)
    hf_ref[...] = hf
    scale = float(D) ** (-0.5)
    d1 = jnp.dot(f_ref[...], hf.T, preferred_element_type=jnp.float32) * scale
    dots_ref[...] = jnp.concatenate([d1, d1, d1], axis=0)
    mx = jnp.max(h, axis=0, keepdims=True)
    eh = jnp.exp(h - mx)
    hs_ref[...] = eh / jnp.sum(eh, axis=0, keepdims=True)


def _tc_fin2(h, out20, f):
    return pl.pallas_call(
        _fin2_body,
        out_shape=[jax.ShapeDtypeStruct((N, M), jnp.float32),
                   jax.ShapeDtypeStruct((M, D), jnp.float32),
                   jax.ShapeDtypeStruct((3 * N, M), jnp.float32)],
    )(h, out20, f)


def kernel(edge_index, features, lin_W, lin_b, gcn1_W, gcn1_b, gcn2_W, gcn2_b,
           lin1_W, lin1_b):
    src = edge_index[0].astype(jnp.int32)
    dst = edge_index[1].astype(jnp.int32)
    pad = jnp.zeros((EP - E,), jnp.int32)
    srcp = jnp.concatenate([src, pad]).reshape(NROWS, 1, CH)
    dstp = jnp.concatenate([dst, pad]).reshape(NROWS, 1, CH)
    srcp64 = srcp.reshape(NROWS, 2, CH // 2)

    _sim_kernel = _make_sim_kernel()
    _agg_kernel = _make_agg_kernel()
    fhat, dstack, xw1 = _tc_prep(features, lin_W, lin_b, gcn1_W)
    idx012, degacc = _sim_kernel(dstack, fhat, srcp, dstp)
    degacc = degacc.reshape(NC, ND, L)
    (p1,) = _tc_p1(degacc, xw1)
    (macc1,) = _agg_kernel(p1.reshape(NC * N, M), srcp64, idx012)
    macc1 = macc1.reshape(NC, ACC_ROWS, M)
    p2, xw2 = _tc_mid(macc1, degacc, xw1, gcn1_b.reshape(1, D), gcn2_W)
    (macc2,) = _agg_kernel(p2.reshape(NC * N, M), srcp64, idx012)
    macc2 = macc2.reshape(NC, ACC_ROWS, M)
    out20, h = _tc_fin1(macc2, degacc, xw2, gcn2_b.reshape(1, D), lin1_W,
                        lin1_b.reshape(1, M))
    hs, hf, dots = _tc_fin2(h, out20, features)
    return (hs, hf, dots)
